# baseline (speedup 1.0000x reference)
import jax
import jax.numpy as jnp
from jax import lax
from jax.experimental import pallas as pl
from jax.experimental.pallas import tpu as pltpu
from jax.experimental.pallas import tpu_sc as plsc

F = 26
B = 16384
V = 100000
D = 16

_INFO = plsc.get_sparse_core_info()
NC = _INFO.num_cores
NS = _INFO.num_subcores
L = _INFO.num_lanes
NW = NC * NS

FD = F * D
RPW = FD // NW
BCH = 4096
NBC = B // BCH


def _body(x_hbm, tab_hbm, out_hbm, row_v, xrow_v, out0_v, out1_v,
          rsem, osem0, osem1):
    c = lax.axis_index("c")
    s = lax.axis_index("s")
    wid = s * NC + c

    z0 = jnp.zeros((L,), jnp.int32)
    outs = (out0_v, out1_v)
    osems = (osem0, osem1)
    pending = [None, None]
    par = 0

    for i in range(RPW):
        fd = wid * RPW + i
        f = fd // D
        d = fd - f * D
        rcp = pltpu.make_async_copy(
            tab_hbm.at[pl.ds(f, 1), pl.ds(d, 1), :], row_v, rsem
        )
        rcp.start()
        if i == 0:
            pltpu.sync_copy(x_hbm.at[pl.ds(f, 1), :], xrow_v)
        else:
            @pl.when(d == 0)
            def _():
                pltpu.sync_copy(x_hbm.at[pl.ds(f, 1), :], xrow_v)
        rcp.wait()

        for cb in range(NBC):
            b0 = cb * BCH
            ov = outs[par]
            if pending[par] is not None:
                pending[par].wait()

            @plsc.parallel_loop(0, BCH, step=L, unroll=16)
            def _gather(g):
                iv = xrow_v[0, pl.ds(b0 + g, L)]
                vals = plsc.load_gather(row_v, [z0, z0, iv])
                ov[0, 0, pl.ds(g, L)] = vals
            ocp = pltpu.make_async_copy(
                ov, out_hbm.at[pl.ds(f, 1), pl.ds(d, 1), pl.ds(b0, BCH)],
                osems[par],
            )
            ocp.start()
            pending[par] = ocp
            par ^= 1

    for q in (0, 1):
        if pending[q] is not None:
            pending[q].wait()


@jax.jit
def kernel(X, tables):
    xt = X.T
    tt = tables.transpose(0, 2, 1)
    mesh = plsc.VectorSubcoreMesh(core_axis_name="c", subcore_axis_name="s")
    out = pl.kernel(
        _body,
        out_type=jax.ShapeDtypeStruct((F, D, B), jnp.float32),
        mesh=mesh,
        compiler_params=pltpu.CompilerParams(needs_layout_passes=False),
        scratch_types=[
            pltpu.VMEM((1, 1, V), jnp.float32),
            pltpu.VMEM((1, B), jnp.int32),
            pltpu.VMEM((1, 1, BCH), jnp.float32),
            pltpu.VMEM((1, 1, BCH), jnp.float32),
            pltpu.SemaphoreType.DMA,
            pltpu.SemaphoreType.DMA,
            pltpu.SemaphoreType.DMA,
        ],
    )(xt, tt)
    return out.transpose(2, 0, 1)


# speedup vs baseline: 1.0296x; 1.0296x over previous
import jax
import jax.numpy as jnp
from jax import lax
from jax.experimental import pallas as pl
from jax.experimental.pallas import tpu as pltpu
from jax.experimental.pallas import tpu_sc as plsc

F = 26
B = 16384
V = 100000
D = 16

_INFO = plsc.get_sparse_core_info()
NC = _INFO.num_cores
NS = _INFO.num_subcores
L = _INFO.num_lanes
NW = NC * NS

FD = F * D
RPW = FD // NW
BCH = 4096
NBC = B // BCH


def _body(x_hbm, tab_hbm, out_hbm, row_v, xrow_v, out0_v, out1_v,
          rsem, osem0, osem1):
    c = lax.axis_index("c")
    s = lax.axis_index("s")
    wid = s * NC + c

    z0 = jnp.zeros((L,), jnp.int32)
    outs = (out0_v, out1_v)
    osems = (osem0, osem1)
    pending = [None, None]
    par = 0

    for i in range(RPW):
        fd = wid * RPW + i
        f = fd // D
        d = fd - f * D
        rcp = pltpu.make_async_copy(
            tab_hbm.at[pl.ds(f, 1), pl.ds(d, 1), :], row_v, rsem
        )
        rcp.start()
        if i == 0:
            pltpu.sync_copy(x_hbm.at[pl.ds(f, 1), :], xrow_v)
        else:
            @pl.when(d == 0)
            def _():
                pltpu.sync_copy(x_hbm.at[pl.ds(f, 1), :], xrow_v)
        rcp.wait()

        for cb in range(NBC):
            b0 = cb * BCH
            ov = outs[par]
            if pending[par] is not None:
                pending[par].wait()

            @plsc.parallel_loop(0, BCH, step=L, unroll=8)
            def _gather(g):
                iv = xrow_v[0, pl.ds(b0 + g, L)]
                vals = plsc.load_gather(row_v, [z0, z0, iv])
                ov[0, 0, pl.ds(g, L)] = vals
            ocp = pltpu.make_async_copy(
                ov, out_hbm.at[pl.ds(f, 1), pl.ds(d, 1), pl.ds(b0, BCH)],
                osems[par],
            )
            ocp.start()
            pending[par] = ocp
            par ^= 1

    for q in (0, 1):
        if pending[q] is not None:
            pending[q].wait()


@jax.jit
def kernel(X, tables):
    xt = X.T
    tt = tables.transpose(0, 2, 1)
    mesh = plsc.VectorSubcoreMesh(core_axis_name="c", subcore_axis_name="s")
    out = pl.kernel(
        _body,
        out_type=jax.ShapeDtypeStruct((F, D, B), jnp.float32),
        mesh=mesh,
        compiler_params=pltpu.CompilerParams(needs_layout_passes=False),
        scratch_types=[
            pltpu.VMEM((1, 1, V), jnp.float32),
            pltpu.VMEM((1, B), jnp.int32),
            pltpu.VMEM((1, 1, BCH), jnp.float32),
            pltpu.VMEM((1, 1, BCH), jnp.float32),
            pltpu.SemaphoreType.DMA,
            pltpu.SemaphoreType.DMA,
            pltpu.SemaphoreType.DMA,
        ],
    )(xt, tt)
    return out.transpose(2, 0, 1)
